# baseline (speedup 1.0000x reference)
import functools

import jax
import jax.numpy as jnp
import numpy as np
from jax import lax
from jax.experimental import pallas as pl
from jax.experimental.pallas import tpu as pltpu
from jax.experimental.pallas import tpu_sc as plsc

E = 1600000
N = 50000
G = 8
EMB = 16

_BLK = 6400


def _mlp_body(vt_ref, xt_ref, w1_ref, b1_ref, w2_ref, b2_ref, out_ref):
    vt = vt_ref[...]
    xt = xt_ref[...]
    a_parts = [vt[a : a + 1, :] * xt for a in range(3)]
    AT = jnp.concatenate(a_parts, axis=0)
    eo_parts = [vt[b : b + 1, :] * AT for b in range(3)]
    EOT = jnp.concatenate(eo_parts, axis=0)
    h = jnp.dot(w1_ref[...], EOT, preferred_element_type=jnp.float32)
    h = h + b1_ref[...]
    h = h * jax.nn.sigmoid(h)
    out = jnp.dot(w2_ref[...], h, preferred_element_type=jnp.float32)
    out_ref[...] = out + b2_ref[...]


def _edge_mlp(vt, xt, w1t, b1c, w2t, b2c):
    grid = (E // _BLK,)
    return pl.pallas_call(
        _mlp_body,
        grid=grid,
        in_specs=[
            pl.BlockSpec((3, _BLK), lambda i: (0, i)),
            pl.BlockSpec((EMB, _BLK), lambda i: (0, i)),
            pl.BlockSpec((144, 144), lambda i: (0, 0)),
            pl.BlockSpec((144, 1), lambda i: (0, 0)),
            pl.BlockSpec((16, 144), lambda i: (0, 0)),
            pl.BlockSpec((16, 1), lambda i: (0, 0)),
        ],
        out_specs=pl.BlockSpec((16, _BLK), lambda i: (0, i)),
        out_shape=jax.ShapeDtypeStruct((16, E), jnp.float32),
        compiler_params=pltpu.CompilerParams(
            dimension_semantics=("arbitrary",),
        ),
    )(vt, xt, w1t, b1c, w2t, b2c)


_PERM = np.empty(144, dtype=np.int32)
for _b in range(3):
    for _a in range(3):
        for _i in range(16):
            _PERM[_b * 48 + _a * 16 + _i] = _i * 9 + _a * 3 + _b


_NB = E // 128
_NW = 32
_SB = 8
_NSB = _NB // _SB
_SB_PER_W = _NSB // _NW
_REM = _NSB - _SB_PER_W * _NW
_TAIL_ROW = _NSB * _SB
_NPS_A = 3128
_NPS_B = N - 15 * _NPS_A


def _sc_scatter_body(idx_hbm, e9t_hbm, out_hbm, idx_v, cols_v, rows_v, accum):
    c = lax.axis_index("c")
    s = lax.axis_index("s")
    wid = s * 2 + c
    iota16 = lax.iota(jnp.int32, 16)

    def _zero(i, _):
        rows_v[i, :] = jnp.zeros((16,), jnp.float32)
        return 0
    lax.fori_loop(0, _SB * 128, _zero, 0)
    my_base = pl.multiple_of(s * _NPS_A, 8)
    for k in range(4):
        off = k * 1024
        size = [1024, 1024, 1024, 56][k]
        size_b = [1024, 1024, 1024, 8][k]
        @pl.when(s < 15)
        def _():
            pltpu.sync_copy(
                rows_v.at[pl.ds(0, size)],
                accum.at[pl.ds(my_base + off, size)],
            )
        @pl.when(s == 15)
        def _():
            pltpu.sync_copy(
                rows_v.at[pl.ds(0, size_b)],
                accum.at[pl.ds(my_base + off, size_b)],
            )
    plsc.subcore_barrier()

    base_sb = wid * _SB_PER_W + jnp.minimum(wid, _REM)

    def _transpose_cols(n_edges):
        def _t(e, colv):
            val = plsc.load_gather(cols_v, [iota16, colv])
            plsc.store_scatter(rows_v, [colv, iota16], val)
            return colv + 1
        lax.fori_loop(0, n_edges, _t, jnp.zeros((16,), jnp.int32))

    def _do_sb(sb):
        row0 = pl.multiple_of(sb * _SB, _SB)
        ecol0 = pl.multiple_of(sb * (_SB * 128), _SB * 128)
        pltpu.sync_copy(idx_hbm.at[pl.ds(row0, _SB)], idx_v)
        pltpu.sync_copy(e9t_hbm.at[:, pl.ds(ecol0, _SB * 128)], cols_v)
        _transpose_cols(_SB * 128)
        for j in range(_SB):
            pltpu.sync_copy(
                rows_v.at[pl.ds(j * 128, 128)],
                accum.at[idx_v.at[j]],
                add=True,
            )

    def _chunk(t, _):
        _do_sb(base_sb + t)
        return 0

    lax.fori_loop(0, _SB_PER_W, _chunk, 0)

    @pl.when(wid < _REM)
    def _():
        _do_sb(base_sb + _SB_PER_W)

    @pl.when(wid == _NW - 1)
    def _():
        pltpu.sync_copy(idx_hbm.at[pl.ds(_TAIL_ROW, 4)], idx_v.at[pl.ds(0, 4)])
        pltpu.sync_copy(
            e9t_hbm.at[:, pl.ds(_TAIL_ROW * 128, 512)],
            cols_v.at[:, pl.ds(0, 512)],
        )
        _transpose_cols(512)
        for j in range(4):
            pltpu.sync_copy(
                rows_v.at[pl.ds(j * 128, 128)],
                accum.at[idx_v.at[j]],
                add=True,
            )

    plsc.subcore_barrier()
    @pl.when(s < 15)
    def _():
        pltpu.sync_copy(
            accum.at[pl.ds(my_base, _NPS_A)],
            out_hbm.at[c].at[pl.ds(my_base, _NPS_A)],
        )
    @pl.when(s == 15)
    def _():
        pltpu.sync_copy(
            accum.at[pl.ds(15 * _NPS_A, _NPS_B)],
            out_hbm.at[c].at[pl.ds(15 * _NPS_A, _NPS_B)],
        )


def _sc_scatter(idx2d, e9t):
    mesh = plsc.VectorSubcoreMesh(core_axis_name="c", subcore_axis_name="s")
    fn = functools.partial(
        pl.kernel,
        mesh=mesh,
        compiler_params=pltpu.CompilerParams(
            use_tc_tiling_on_sc=False, needs_layout_passes=False
        ),
        out_type=jax.ShapeDtypeStruct((2, N, 16), jnp.float32),
        scratch_types=[
            pltpu.VMEM((_SB, 128), jnp.int32),
            pltpu.VMEM((16, _SB * 128), jnp.float32),
            pltpu.VMEM((_SB * 128, 16), jnp.float32),
            pltpu.VMEM_SHARED((N, 16), jnp.float32),
        ],
    )(_sc_scatter_body)
    return fn(idx2d, e9t)


_NBLK3 = 10
_B3 = N // _NBLK3


def _finish_body(part_ref, bat_ref, out_ref, gsum, gcnt):
    i = pl.program_id(0)

    @pl.when(i == 0)
    def _():
        gsum[...] = jnp.zeros((G, 16), jnp.float32)
        gcnt[...] = jnp.zeros((G, 16), jnp.float32)

    sums = part_ref[0] + part_ref[1]
    cnt = jnp.maximum(sums[:, 9:10], 1.0)
    node = sums / cnt
    bat = bat_ref[...]
    for g in range(G):
        m = (bat == g)
        gsum[g : g + 1, :] += jnp.sum(
            jnp.where(m, node, 0.0), axis=0, keepdims=True
        )
        gcnt[g : g + 1, :] += jnp.broadcast_to(
            jnp.sum(m.astype(jnp.float32)), (1, 16)
        )

    @pl.when(i == _NBLK3 - 1)
    def _():
        out_ref[...] = gsum[...] / jnp.maximum(gcnt[...], 1.0)


def _finish(partials, bat2d):
    return pl.pallas_call(
        _finish_body,
        grid=(_NBLK3,),
        in_specs=[
            pl.BlockSpec((2, _B3, 16), lambda i: (0, i, 0)),
            pl.BlockSpec((_B3, 1), lambda i: (i, 0)),
        ],
        out_specs=pl.BlockSpec((G, 16), lambda i: (0, 0)),
        out_shape=jax.ShapeDtypeStruct((G, 16), jnp.float32),
        scratch_shapes=[
            pltpu.VMEM((G, 16), jnp.float32),
            pltpu.VMEM((G, 16), jnp.float32),
        ],
        compiler_params=pltpu.CompilerParams(
            dimension_semantics=("arbitrary",),
        ),
    )(partials, bat2d)


def kernel(edge_distance_vec, x_edge, edge_index, batch, W1, b1, W2, b2):
    idx2d = edge_index.astype(jnp.int32).reshape(_NB, 128)
    w1t = W1[jnp.asarray(_PERM), :].T
    b1c = b1.reshape(144, 1)
    w2t = jnp.pad(W2, ((0, 0), (0, 16 - 9))).T
    b2c = jnp.pad(b2, (0, 16 - 9)).at[9].set(1.0).reshape(16, 1)

    e9t = _edge_mlp(edge_distance_vec.T, x_edge.T, w1t, b1c, w2t, b2c)
    partials = _sc_scatter(idx2d, e9t)
    bat2d = batch.astype(jnp.int32).reshape(N, 1)
    stress = _finish(partials, bat2d)
    return stress[:, :9]


# speedup vs baseline: 2.3837x; 2.3837x over previous
import functools

import jax
import jax.numpy as jnp
import numpy as np
from jax import lax
from jax.experimental import pallas as pl
from jax.experimental.pallas import tpu as pltpu
from jax.experimental.pallas import tpu_sc as plsc

E = 1600000
N = 50000
G = 8
EMB = 16

_BLK = 6400


def _mlp_body(vt_ref, xt_ref, w1_ref, b1_ref, w2_ref, b2_ref, out_ref):
    vt = vt_ref[...]
    xt = xt_ref[...]
    a_parts = [vt[a : a + 1, :] * xt for a in range(3)]
    AT = jnp.concatenate(a_parts, axis=0)
    eo_parts = [vt[b : b + 1, :] * AT for b in range(3)]
    EOT = jnp.concatenate(eo_parts, axis=0)
    h = jnp.dot(w1_ref[...], EOT, preferred_element_type=jnp.float32)
    h = h + b1_ref[...]
    h = h * jax.nn.sigmoid(h)
    out = jnp.dot(w2_ref[...], h, preferred_element_type=jnp.float32)
    out_ref[...] = (out + b2_ref[...]).T


def _edge_mlp(vt, xt, w1t, b1c, w2t, b2c):
    grid = (E // _BLK,)
    return pl.pallas_call(
        _mlp_body,
        grid=grid,
        in_specs=[
            pl.BlockSpec((3, _BLK), lambda i: (0, i)),
            pl.BlockSpec((EMB, _BLK), lambda i: (0, i)),
            pl.BlockSpec((144, 144), lambda i: (0, 0)),
            pl.BlockSpec((144, 1), lambda i: (0, 0)),
            pl.BlockSpec((16, 144), lambda i: (0, 0)),
            pl.BlockSpec((16, 1), lambda i: (0, 0)),
        ],
        out_specs=pl.BlockSpec((_BLK, 16), lambda i: (i, 0)),
        out_shape=jax.ShapeDtypeStruct((E, 16), jnp.float32),
        compiler_params=pltpu.CompilerParams(
            dimension_semantics=("arbitrary",),
        ),
    )(vt, xt, w1t, b1c, w2t, b2c)


_PERM = np.empty(144, dtype=np.int32)
for _b in range(3):
    for _a in range(3):
        for _i in range(16):
            _PERM[_b * 48 + _a * 16 + _i] = _i * 9 + _a * 3 + _b


_NB = E // 128
_NW = 32
_SB = 8
_NSB = _NB // _SB
_SB_PER_W = _NSB // _NW
_REM = _NSB - _SB_PER_W * _NW
_TAIL_ROW = _NSB * _SB
_NPS_A = 3128
_NPS_B = N - 15 * _NPS_A


def _sc_scatter_body(idx_hbm, e9_hbm, out_hbm, idx_v, rows_v, accum):
    c = lax.axis_index("c")
    s = lax.axis_index("s")
    wid = s * 2 + c

    def _zero(i, _):
        rows_v[i, :] = jnp.zeros((16,), jnp.float32)
        return 0
    lax.fori_loop(0, _SB * 128, _zero, 0)
    my_base = pl.multiple_of(s * _NPS_A, 8)
    for k in range(4):
        off = k * 1024
        size = [1024, 1024, 1024, 56][k]
        size_b = [1024, 1024, 1024, 8][k]
        @pl.when(s < 15)
        def _():
            pltpu.sync_copy(
                rows_v.at[pl.ds(0, size)],
                accum.at[pl.ds(my_base + off, size)],
            )
        @pl.when(s == 15)
        def _():
            pltpu.sync_copy(
                rows_v.at[pl.ds(0, size_b)],
                accum.at[pl.ds(my_base + off, size_b)],
            )
    plsc.subcore_barrier()

    base_sb = wid * _SB_PER_W + jnp.minimum(wid, _REM)

    def _do_sb(sb):
        row0 = pl.multiple_of(sb * _SB, _SB)
        erow0 = pl.multiple_of(sb * (_SB * 128), _SB * 128)
        pltpu.sync_copy(idx_hbm.at[pl.ds(row0, _SB)], idx_v)
        pltpu.sync_copy(e9_hbm.at[pl.ds(erow0, _SB * 128)], rows_v)
        for j in range(_SB):
            pltpu.sync_copy(
                rows_v.at[pl.ds(j * 128, 128)],
                accum.at[idx_v.at[j]],
                add=True,
            )

    def _chunk(t, _):
        _do_sb(base_sb + t)
        return 0

    lax.fori_loop(0, _SB_PER_W, _chunk, 0)

    @pl.when(wid < _REM)
    def _():
        _do_sb(base_sb + _SB_PER_W)

    @pl.when(wid == _NW - 1)
    def _():
        pltpu.sync_copy(idx_hbm.at[pl.ds(_TAIL_ROW, 4)], idx_v.at[pl.ds(0, 4)])
        pltpu.sync_copy(
            e9_hbm.at[pl.ds(_TAIL_ROW * 128, 512)], rows_v.at[pl.ds(0, 512)]
        )
        for j in range(4):
            pltpu.sync_copy(
                rows_v.at[pl.ds(j * 128, 128)],
                accum.at[idx_v.at[j]],
                add=True,
            )

    plsc.subcore_barrier()
    @pl.when(s < 15)
    def _():
        pltpu.sync_copy(
            accum.at[pl.ds(my_base, _NPS_A)],
            out_hbm.at[c].at[pl.ds(my_base, _NPS_A)],
        )
    @pl.when(s == 15)
    def _():
        pltpu.sync_copy(
            accum.at[pl.ds(15 * _NPS_A, _NPS_B)],
            out_hbm.at[c].at[pl.ds(15 * _NPS_A, _NPS_B)],
        )


def _sc_scatter(idx2d, e9):
    mesh = plsc.VectorSubcoreMesh(core_axis_name="c", subcore_axis_name="s")
    fn = functools.partial(
        pl.kernel,
        mesh=mesh,
        compiler_params=pltpu.CompilerParams(use_tc_tiling_on_sc=False),
        out_type=jax.ShapeDtypeStruct((2, N, 16), jnp.float32),
        scratch_types=[
            pltpu.VMEM((_SB, 128), jnp.int32),
            pltpu.VMEM((_SB * 128, 16), jnp.float32),
            pltpu.VMEM_SHARED((N, 16), jnp.float32),
        ],
    )(_sc_scatter_body)
    return fn(idx2d, e9)


_NBLK3 = 10
_B3 = N // _NBLK3


def _finish_body(part_ref, bat_ref, out_ref, gsum, gcnt):
    i = pl.program_id(0)

    @pl.when(i == 0)
    def _():
        gsum[...] = jnp.zeros((G, 16), jnp.float32)
        gcnt[...] = jnp.zeros((G, 16), jnp.float32)

    sums = part_ref[0] + part_ref[1]
    cnt = jnp.maximum(sums[:, 9:10], 1.0)
    node = sums / cnt
    bat = bat_ref[...]
    for g in range(G):
        m = (bat == g)
        gsum[g : g + 1, :] += jnp.sum(
            jnp.where(m, node, 0.0), axis=0, keepdims=True
        )
        gcnt[g : g + 1, :] += jnp.broadcast_to(
            jnp.sum(m.astype(jnp.float32)), (1, 16)
        )

    @pl.when(i == _NBLK3 - 1)
    def _():
        out_ref[...] = gsum[...] / jnp.maximum(gcnt[...], 1.0)


def _finish(partials, bat2d):
    return pl.pallas_call(
        _finish_body,
        grid=(_NBLK3,),
        in_specs=[
            pl.BlockSpec((2, _B3, 16), lambda i: (0, i, 0)),
            pl.BlockSpec((_B3, 1), lambda i: (i, 0)),
        ],
        out_specs=pl.BlockSpec((G, 16), lambda i: (0, 0)),
        out_shape=jax.ShapeDtypeStruct((G, 16), jnp.float32),
        scratch_shapes=[
            pltpu.VMEM((G, 16), jnp.float32),
            pltpu.VMEM((G, 16), jnp.float32),
        ],
        compiler_params=pltpu.CompilerParams(
            dimension_semantics=("arbitrary",),
        ),
    )(partials, bat2d)


def kernel(edge_distance_vec, x_edge, edge_index, batch, W1, b1, W2, b2):
    idx2d = edge_index.astype(jnp.int32).reshape(_NB, 128)
    w1t = W1[jnp.asarray(_PERM), :].T
    b1c = b1.reshape(144, 1)
    w2t = jnp.pad(W2, ((0, 0), (0, 16 - 9))).T
    b2c = jnp.pad(b2, (0, 16 - 9)).at[9].set(1.0).reshape(16, 1)

    e9t = _edge_mlp(edge_distance_vec.T, x_edge.T, w1t, b1c, w2t, b2c)
    partials = _sc_scatter(idx2d, e9t)
    bat2d = batch.astype(jnp.int32).reshape(N, 1)
    stress = _finish(partials, bat2d)
    return stress[:, :9]


# speedup vs baseline: 2.8683x; 1.2033x over previous
import functools

import jax
import jax.numpy as jnp
import numpy as np
from jax import lax
from jax.experimental import pallas as pl
from jax.experimental.pallas import tpu as pltpu
from jax.experimental.pallas import tpu_sc as plsc

E = 1600000
N = 50000
G = 8
EMB = 16

_BLK = 6400


def _mlp_body(vt_ref, xt_ref, w1_ref, b1_ref, w2_ref, b2_ref, out_ref):
    vt = vt_ref[...]
    xt = xt_ref[...]
    a_parts = [vt[a : a + 1, :] * xt for a in range(3)]
    AT = jnp.concatenate(a_parts, axis=0)
    eo_parts = [vt[b : b + 1, :] * AT for b in range(3)]
    EOT = jnp.concatenate(eo_parts, axis=0)
    h = jnp.dot(w1_ref[...], EOT, preferred_element_type=jnp.float32)
    h = h + b1_ref[...]
    h = h * jax.nn.sigmoid(h)
    out = jnp.dot(w2_ref[...], h, preferred_element_type=jnp.float32)
    out16 = (out + b2_ref[...]).T
    c = _BLK // 8
    out_ref[...] = jnp.concatenate(
        [out16[s * c : (s + 1) * c, :] for s in range(8)], axis=1
    )


def _edge_mlp(vt, xt, w1t, b1c, w2t, b2c):
    grid = (E // _BLK,)
    return pl.pallas_call(
        _mlp_body,
        grid=grid,
        in_specs=[
            pl.BlockSpec((3, _BLK), lambda i: (0, i)),
            pl.BlockSpec((EMB, _BLK), lambda i: (0, i)),
            pl.BlockSpec((144, 144), lambda i: (0, 0)),
            pl.BlockSpec((144, 1), lambda i: (0, 0)),
            pl.BlockSpec((16, 144), lambda i: (0, 0)),
            pl.BlockSpec((16, 1), lambda i: (0, 0)),
        ],
        out_specs=pl.BlockSpec((_BLK // 8, 128), lambda i: (i, 0)),
        out_shape=jax.ShapeDtypeStruct((E // 8, 128), jnp.float32),
        compiler_params=pltpu.CompilerParams(
            dimension_semantics=("arbitrary",),
        ),
    )(vt, xt, w1t, b1c, w2t, b2c)


_PERM = np.empty(144, dtype=np.int32)
for _b in range(3):
    for _a in range(3):
        for _i in range(16):
            _PERM[_b * 48 + _a * 16 + _i] = _i * 9 + _a * 3 + _b


_NB = E // 128
_NW = 32
_SB = 8
_NSB = _NB // _SB
_SB_PER_W = _NSB // _NW
_REM = _NSB - _SB_PER_W * _NW
_TAIL_ROW = _NSB * _SB
_NPS_A = 3128
_NPS_B = N - 15 * _NPS_A


def _sc_scatter_body(idx_hbm, e9_hbm, out_hbm, idx_v, rows_v, accum):
    c = lax.axis_index("c")
    s = lax.axis_index("s")
    wid = s * 2 + c
    rows16 = rows_v
    e16_hbm = e9_hbm

    def _zero(i, _):
        rows16[i, :] = jnp.zeros((16,), jnp.float32)
        return 0
    lax.fori_loop(0, _SB * 128, _zero, 0)
    my_base = pl.multiple_of(s * _NPS_A, 8)
    for k in range(4):
        off = k * 1024
        size = [1024, 1024, 1024, 56][k]
        size_b = [1024, 1024, 1024, 8][k]
        @pl.when(s < 15)
        def _():
            pltpu.sync_copy(
                rows16.at[pl.ds(0, size)],
                accum.at[pl.ds(my_base + off, size)],
            )
        @pl.when(s == 15)
        def _():
            pltpu.sync_copy(
                rows16.at[pl.ds(0, size_b)],
                accum.at[pl.ds(my_base + off, size_b)],
            )
    plsc.subcore_barrier()

    base_sb = wid * _SB_PER_W + jnp.minimum(wid, _REM)

    def _do_sb(sb):
        row0 = pl.multiple_of(sb * _SB, _SB)
        erow0 = pl.multiple_of(sb * (_SB * 128), _SB * 128)
        pltpu.sync_copy(idx_hbm.at[pl.ds(row0, _SB)], idx_v)
        pltpu.sync_copy(e16_hbm.at[pl.ds(erow0, _SB * 128)], rows_v)
        for j in range(_SB):
            pltpu.sync_copy(
                rows16.at[pl.ds(j * 128, 128)],
                accum.at[idx_v.at[j]],
                add=True,
            )

    def _chunk(t, _):
        _do_sb(base_sb + t)
        return 0

    lax.fori_loop(0, _SB_PER_W, _chunk, 0)

    @pl.when(wid < _REM)
    def _():
        _do_sb(base_sb + _SB_PER_W)

    @pl.when(wid == _NW - 1)
    def _():
        pltpu.sync_copy(idx_hbm.at[pl.ds(_TAIL_ROW, 4)], idx_v.at[pl.ds(0, 4)])
        pltpu.sync_copy(
            e16_hbm.at[pl.ds(_TAIL_ROW * 128, 512)], rows_v.at[pl.ds(0, 512)]
        )
        for j in range(4):
            pltpu.sync_copy(
                rows16.at[pl.ds(j * 128, 128)],
                accum.at[idx_v.at[j]],
                add=True,
            )

    plsc.subcore_barrier()
    @pl.when(s < 15)
    def _():
        pltpu.sync_copy(
            accum.at[pl.ds(my_base, _NPS_A)],
            out_hbm.at[c].at[pl.ds(my_base, _NPS_A)],
        )
    @pl.when(s == 15)
    def _():
        pltpu.sync_copy(
            accum.at[pl.ds(15 * _NPS_A, _NPS_B)],
            out_hbm.at[c].at[pl.ds(15 * _NPS_A, _NPS_B)],
        )


def _sc_scatter(idx2d, e9):
    mesh = plsc.VectorSubcoreMesh(core_axis_name="c", subcore_axis_name="s")
    fn = functools.partial(
        pl.kernel,
        mesh=mesh,
        compiler_params=pltpu.CompilerParams(use_tc_tiling_on_sc=False),
        out_type=jax.ShapeDtypeStruct((2, N, 16), jnp.float32),
        scratch_types=[
            pltpu.VMEM((_SB, 128), jnp.int32),
            pltpu.VMEM((_SB * 128, 16), jnp.float32),
            pltpu.VMEM_SHARED((N, 16), jnp.float32),
        ],
    )(_sc_scatter_body)
    return fn(idx2d, e9)


_NBLK3 = 10
_B3 = N // _NBLK3


def _finish_body(part_ref, bat_ref, out_ref, gsum, gcnt):
    i = pl.program_id(0)

    @pl.when(i == 0)
    def _():
        gsum[...] = jnp.zeros((G, 16), jnp.float32)
        gcnt[...] = jnp.zeros((G, 16), jnp.float32)

    sums = part_ref[0] + part_ref[1]
    cnt = jnp.maximum(sums[:, 9:10], 1.0)
    node = sums / cnt
    bat = bat_ref[...]
    for g in range(G):
        m = (bat == g)
        gsum[g : g + 1, :] += jnp.sum(
            jnp.where(m, node, 0.0), axis=0, keepdims=True
        )
        gcnt[g : g + 1, :] += jnp.broadcast_to(
            jnp.sum(m.astype(jnp.float32)), (1, 16)
        )

    @pl.when(i == _NBLK3 - 1)
    def _():
        out_ref[...] = gsum[...] / jnp.maximum(gcnt[...], 1.0)


def _finish(partials, bat2d):
    return pl.pallas_call(
        _finish_body,
        grid=(_NBLK3,),
        in_specs=[
            pl.BlockSpec((2, _B3, 16), lambda i: (0, i, 0)),
            pl.BlockSpec((_B3, 1), lambda i: (i, 0)),
        ],
        out_specs=pl.BlockSpec((G, 16), lambda i: (0, 0)),
        out_shape=jax.ShapeDtypeStruct((G, 16), jnp.float32),
        scratch_shapes=[
            pltpu.VMEM((G, 16), jnp.float32),
            pltpu.VMEM((G, 16), jnp.float32),
        ],
        compiler_params=pltpu.CompilerParams(
            dimension_semantics=("arbitrary",),
        ),
    )(partials, bat2d)


def kernel(edge_distance_vec, x_edge, edge_index, batch, W1, b1, W2, b2):
    idx2d = (
        edge_index.astype(jnp.int32)
        .reshape(E // _BLK, 8, _BLK // 8)
        .transpose(0, 2, 1)
        .reshape(_NB, 128)
    )
    w1t = W1[jnp.asarray(_PERM), :].T
    b1c = b1.reshape(144, 1)
    w2t = jnp.pad(W2, ((0, 0), (0, 16 - 9))).T
    b2c = jnp.pad(b2, (0, 16 - 9)).at[9].set(1.0).reshape(16, 1)

    e9p = _edge_mlp(edge_distance_vec.T, x_edge.T, w1t, b1c, w2t, b2c)
    partials = _sc_scatter(idx2d, e9p.reshape(E, 16))
    bat2d = batch.astype(jnp.int32).reshape(N, 1)
    stress = _finish(partials, bat2d)
    return stress[:, :9]


# speedup vs baseline: 3.1262x; 1.0899x over previous
import functools

import jax
import jax.numpy as jnp
import numpy as np
from jax import lax
from jax.experimental import pallas as pl
from jax.experimental.pallas import tpu as pltpu
from jax.experimental.pallas import tpu_sc as plsc

E = 1600000
N = 50000
G = 8
EMB = 16

_BLK = 6400


def _mlp_body(vt_ref, xt_ref, w1_ref, b1_ref, w2_ref, b2_ref, out_ref):
    vt = vt_ref[...]
    xt = xt_ref[...]
    a_parts = [vt[a : a + 1, :] * xt for a in range(3)]
    AT = jnp.concatenate(a_parts, axis=0)
    eo_parts = [vt[b : b + 1, :] * AT for b in range(3)]
    EOT = jnp.concatenate(eo_parts, axis=0)
    h = jnp.dot(w1_ref[...], EOT, preferred_element_type=jnp.float32)
    h = h + b1_ref[...]
    h = h * jax.nn.sigmoid(h)
    out16 = lax.dot_general(
        h, w2_ref[...], (((0,), (0,)), ((), ())),
        preferred_element_type=jnp.float32,
    )
    out16 = out16 + b2_ref[...]
    c = _BLK // 8
    out_ref[...] = jnp.concatenate(
        [out16[s * c : (s + 1) * c, :] for s in range(8)], axis=1
    )


def _edge_mlp(vt, xt, w1t, b1c, w2t, b2c):
    grid = (E // _BLK,)
    return pl.pallas_call(
        _mlp_body,
        grid=grid,
        in_specs=[
            pl.BlockSpec((3, _BLK), lambda i: (0, i)),
            pl.BlockSpec((EMB, _BLK), lambda i: (0, i)),
            pl.BlockSpec((144, 144), lambda i: (0, 0)),
            pl.BlockSpec((144, 1), lambda i: (0, 0)),
            pl.BlockSpec((144, 16), lambda i: (0, 0)),
            pl.BlockSpec((1, 16), lambda i: (0, 0)),
        ],
        out_specs=pl.BlockSpec((_BLK // 8, 128), lambda i: (i, 0)),
        out_shape=jax.ShapeDtypeStruct((E // 8, 128), jnp.float32),
        compiler_params=pltpu.CompilerParams(
            dimension_semantics=("arbitrary",),
            fuse_transposed_lhs_in_matmul=True,
        ),
    )(vt, xt, w1t, b1c, w2t, b2c)


_PERM = np.empty(144, dtype=np.int32)
for _b in range(3):
    for _a in range(3):
        for _i in range(16):
            _PERM[_b * 48 + _a * 16 + _i] = _i * 9 + _a * 3 + _b


_NB = E // 128
_NW = 32
_SB = 8
_NSB = _NB // _SB
_SB_PER_W = _NSB // _NW
_REM = _NSB - _SB_PER_W * _NW
_TAIL_ROW = _NSB * _SB
_NPS_A = 3128
_NPS_B = N - 15 * _NPS_A


def _sc_scatter_body(idx_hbm, e9_hbm, out_hbm, idx_v, rows_v, accum):
    c = lax.axis_index("c")
    s = lax.axis_index("s")
    wid = s * 2 + c
    rows16 = rows_v
    e16_hbm = e9_hbm

    def _zero(i, _):
        rows16[i, :] = jnp.zeros((16,), jnp.float32)
        return 0
    lax.fori_loop(0, _SB * 128, _zero, 0)
    my_base = pl.multiple_of(s * _NPS_A, 8)
    for k in range(4):
        off = k * 1024
        size = [1024, 1024, 1024, 56][k]
        size_b = [1024, 1024, 1024, 8][k]
        @pl.when(s < 15)
        def _():
            pltpu.sync_copy(
                rows16.at[pl.ds(0, size)],
                accum.at[pl.ds(my_base + off, size)],
            )
        @pl.when(s == 15)
        def _():
            pltpu.sync_copy(
                rows16.at[pl.ds(0, size_b)],
                accum.at[pl.ds(my_base + off, size_b)],
            )
    plsc.subcore_barrier()

    base_sb = wid * _SB_PER_W + jnp.minimum(wid, _REM)

    def _do_sb(sb):
        row0 = pl.multiple_of(sb * _SB, _SB)
        erow0 = pl.multiple_of(sb * (_SB * 128), _SB * 128)
        pltpu.sync_copy(idx_hbm.at[pl.ds(row0, _SB)], idx_v)
        pltpu.sync_copy(e16_hbm.at[pl.ds(erow0, _SB * 128)], rows_v)
        for j in range(_SB):
            pltpu.sync_copy(
                rows16.at[pl.ds(j * 128, 128)],
                accum.at[idx_v.at[j]],
                add=True,
            )

    def _chunk(t, _):
        _do_sb(base_sb + t)
        return 0

    lax.fori_loop(0, _SB_PER_W, _chunk, 0)

    @pl.when(wid < _REM)
    def _():
        _do_sb(base_sb + _SB_PER_W)

    @pl.when(wid == _NW - 1)
    def _():
        pltpu.sync_copy(idx_hbm.at[pl.ds(_TAIL_ROW, 4)], idx_v.at[pl.ds(0, 4)])
        pltpu.sync_copy(
            e16_hbm.at[pl.ds(_TAIL_ROW * 128, 512)], rows_v.at[pl.ds(0, 512)]
        )
        for j in range(4):
            pltpu.sync_copy(
                rows16.at[pl.ds(j * 128, 128)],
                accum.at[idx_v.at[j]],
                add=True,
            )

    plsc.subcore_barrier()
    @pl.when(s < 15)
    def _():
        pltpu.sync_copy(
            accum.at[pl.ds(my_base, _NPS_A)],
            out_hbm.at[c].at[pl.ds(my_base, _NPS_A)],
        )
    @pl.when(s == 15)
    def _():
        pltpu.sync_copy(
            accum.at[pl.ds(15 * _NPS_A, _NPS_B)],
            out_hbm.at[c].at[pl.ds(15 * _NPS_A, _NPS_B)],
        )


def _sc_scatter(idx2d, e9):
    mesh = plsc.VectorSubcoreMesh(core_axis_name="c", subcore_axis_name="s")
    fn = functools.partial(
        pl.kernel,
        mesh=mesh,
        compiler_params=pltpu.CompilerParams(use_tc_tiling_on_sc=False),
        out_type=jax.ShapeDtypeStruct((2, N, 16), jnp.float32),
        scratch_types=[
            pltpu.VMEM((_SB, 128), jnp.int32),
            pltpu.VMEM((_SB * 128, 16), jnp.float32),
            pltpu.VMEM_SHARED((N, 16), jnp.float32),
        ],
    )(_sc_scatter_body)
    return fn(idx2d, e9)


_NBLK3 = 10
_B3 = N // _NBLK3


def _finish_body(part_ref, bat_ref, out_ref, gsum, gcnt):
    i = pl.program_id(0)

    @pl.when(i == 0)
    def _():
        gsum[...] = jnp.zeros((G, 16), jnp.float32)
        gcnt[...] = jnp.zeros((G, 16), jnp.float32)

    sums = part_ref[0] + part_ref[1]
    cnt = jnp.maximum(sums[:, 9:10], 1.0)
    node = sums / cnt
    bat = bat_ref[...]
    for g in range(G):
        m = (bat == g)
        gsum[g : g + 1, :] += jnp.sum(
            jnp.where(m, node, 0.0), axis=0, keepdims=True
        )
        gcnt[g : g + 1, :] += jnp.broadcast_to(
            jnp.sum(m.astype(jnp.float32)), (1, 16)
        )

    @pl.when(i == _NBLK3 - 1)
    def _():
        out_ref[...] = gsum[...] / jnp.maximum(gcnt[...], 1.0)


def _finish(partials, bat2d):
    return pl.pallas_call(
        _finish_body,
        grid=(_NBLK3,),
        in_specs=[
            pl.BlockSpec((2, _B3, 16), lambda i: (0, i, 0)),
            pl.BlockSpec((_B3, 1), lambda i: (i, 0)),
        ],
        out_specs=pl.BlockSpec((G, 16), lambda i: (0, 0)),
        out_shape=jax.ShapeDtypeStruct((G, 16), jnp.float32),
        scratch_shapes=[
            pltpu.VMEM((G, 16), jnp.float32),
            pltpu.VMEM((G, 16), jnp.float32),
        ],
        compiler_params=pltpu.CompilerParams(
            dimension_semantics=("arbitrary",),
        ),
    )(partials, bat2d)


def kernel(edge_distance_vec, x_edge, edge_index, batch, W1, b1, W2, b2):
    idx2d = (
        edge_index.astype(jnp.int32)
        .reshape(E // _BLK, 8, _BLK // 8)
        .transpose(0, 2, 1)
        .reshape(_NB, 128)
    )
    w1t = W1[jnp.asarray(_PERM), :].T
    b1c = b1.reshape(144, 1)
    w2t = jnp.pad(W2, ((0, 0), (0, 16 - 9)))
    b2c = jnp.pad(b2, (0, 16 - 9)).at[9].set(1.0).reshape(1, 16)

    e9p = _edge_mlp(edge_distance_vec.T, x_edge.T, w1t, b1c, w2t, b2c)
    partials = _sc_scatter(idx2d, e9p.reshape(E, 16))
    bat2d = batch.astype(jnp.int32).reshape(N, 1)
    stress = _finish(partials, bat2d)
    return stress[:, :9]
